# baseline (speedup 1.0000x reference)
import functools

import jax
import jax.numpy as jnp
from jax import lax
from jax.experimental import pallas as pl
from jax.experimental.pallas import tpu as pltpu
from jax.experimental.pallas import tpu_sc as plsc

E = 8
TOP_K = 4
_NN = (((1,), (0,)), ((), ()))


def _route_body(x_ref, wgt_ref, bias_ref, coef_ref, *, T):
    xb = x_ref[...].astype(jnp.bfloat16)
    g = lax.dot_general(xb, wgt_ref[...], _NN,
                        preferred_element_type=jnp.float32) + bias_ref[...]
    lane = lax.broadcasted_iota(jnp.int32, (T, E), 1)
    cnt = jnp.zeros((T, E), jnp.int32)
    for jj in range(E):
        gj = g[:, jj:jj + 1]
        above = (gj > g) | ((gj == g) & (jj < lane))
        cnt = cnt + above.astype(jnp.int32)
    sel = cnt < TOP_K
    m = jnp.max(g, axis=1, keepdims=True)
    p = jnp.where(sel, jnp.exp(g - m), 0.0)
    coef_ref[...] = p / jnp.sum(p, axis=1, keepdims=True)


def _routing(xf, Wg, bias):
    N, D = xf.shape
    T = min(1024, N)
    return pl.pallas_call(
        functools.partial(_route_body, T=T),
        grid=(N // T,),
        in_specs=[
            pl.BlockSpec((T, D), lambda tb: (tb, 0)),
            pl.BlockSpec((D, E), lambda tb: (0, 0)),
            pl.BlockSpec((1, E), lambda tb: (0, 0)),
        ],
        out_specs=pl.BlockSpec((T, E), lambda tb: (tb, 0)),
        out_shape=jax.ShapeDtypeStruct((N, E), jnp.float32),
    )(xf, Wg.T.astype(jnp.bfloat16), bias.reshape(1, E).astype(jnp.float32))


def _sc_gather(R, C, dtype):
    mesh = plsc.VectorSubcoreMesh(core_axis_name="c", subcore_axis_name="s")
    rows_per_w = R // 32
    CH = 128
    n_ch = rows_per_w // CH

    @functools.partial(
        pl.kernel, mesh=mesh,
        out_type=jax.ShapeDtypeStruct((R, C), dtype),
        scratch_types=[
            pltpu.VMEM((CH,), jnp.int32),
            pltpu.VMEM((CH, C), dtype),
            pltpu.SemaphoreType.DMA,
        ],
    )
    def gather_k(table_hbm, idx_hbm, out_hbm, idx_v, rows_v, sem):
        wid = lax.axis_index("s") * 2 + lax.axis_index("c")
        base = wid * rows_per_w

        def chunk(g, carry):
            off = base + g * CH
            pltpu.sync_copy(idx_hbm.at[pl.ds(off, CH)], idx_v)
            pltpu.async_copy(table_hbm.at[idx_v], rows_v, sem).wait()
            pltpu.sync_copy(rows_v, out_hbm.at[pl.ds(off, CH)])
            return carry

        lax.fori_loop(0, n_ch, chunk, 0)

    return gather_k


def _group_body(be_ref, xs_ref, w1_ref, w2_ref, w3_ref, y_ref):
    xb = xs_ref[...]
    h = lax.dot_general(xb, w1_ref[0], _NN, preferred_element_type=jnp.float32)
    h = h * jax.nn.sigmoid(h)
    v = lax.dot_general(xb, w2_ref[0], _NN, preferred_element_type=jnp.float32)
    hv = (h * v).astype(jnp.bfloat16)
    y_ref[...] = lax.dot_general(hv, w3_ref[0], _NN,
                                 preferred_element_type=jnp.float32
                                 ).astype(jnp.bfloat16)


def _grouped(x_sorted, block_expert, W1T, W2T, W3T, BT, NBLK):
    _, D = x_sorted.shape
    _, _, H = W1T.shape
    grid_spec = pltpu.PrefetchScalarGridSpec(
        num_scalar_prefetch=1,
        grid=(NBLK,),
        in_specs=[
            pl.BlockSpec((BT, D), lambda i, be: (i, 0)),
            pl.BlockSpec((1, D, H), lambda i, be: (be[i], 0, 0)),
            pl.BlockSpec((1, D, H), lambda i, be: (be[i], 0, 0)),
            pl.BlockSpec((1, H, D), lambda i, be: (be[i], 0, 0)),
        ],
        out_specs=pl.BlockSpec((BT, D), lambda i, be: (i, 0)),
    )
    return pl.pallas_call(
        _group_body,
        grid_spec=grid_spec,
        out_shape=jax.ShapeDtypeStruct((NBLK * BT, D), jnp.bfloat16),
        compiler_params=pltpu.CompilerParams(
            dimension_semantics=("arbitrary",)),
    )(block_expert, x_sorted, W1T, W2T, W3T)


def _comb_body(x_ref, wsa_ref, wsb_ref, wsc_ref, yg_ref, w4_ref, out_ref):
    xb = x_ref[...].astype(jnp.bfloat16)
    h = lax.dot_general(xb, wsa_ref[...], _NN,
                        preferred_element_type=jnp.float32)
    h = h * jax.nn.sigmoid(h)
    v = lax.dot_general(xb, wsb_ref[...], _NN,
                        preferred_element_type=jnp.float32)
    hv = (h * v).astype(jnp.bfloat16)
    acc = lax.dot_general(hv, wsc_ref[...], _NN,
                          preferred_element_type=jnp.float32)
    for s in range(TOP_K):
        acc = acc + yg_ref[s].astype(jnp.float32) * w4_ref[:, s:s + 1]
    out_ref[...] = acc


def _combine(xf, WsaT, WsbT, WscT, yg, w4):
    N, D = xf.shape
    _, H = WsaT.shape
    T = min(512, N)
    return pl.pallas_call(
        _comb_body,
        grid=(N // T,),
        in_specs=[
            pl.BlockSpec((T, D), lambda tb: (tb, 0)),
            pl.BlockSpec((D, H), lambda tb: (0, 0)),
            pl.BlockSpec((D, H), lambda tb: (0, 0)),
            pl.BlockSpec((H, D), lambda tb: (0, 0)),
            pl.BlockSpec((TOP_K, T, D), lambda tb: (0, tb, 0)),
            pl.BlockSpec((T, TOP_K), lambda tb: (tb, 0)),
        ],
        out_specs=pl.BlockSpec((T, D), lambda tb: (tb, 0)),
        out_shape=jax.ShapeDtypeStruct((N, D), jnp.float32),
    )(xf, WsaT, WsbT, WscT, yg, w4)


def kernel(x, Wg, W1, W2, W3, Ws1, Ws2, Ws3, routing_bias):
    B, S, D = x.shape
    _, H, _ = W1.shape
    N = B * S
    BT = 512
    NBLK = (TOP_K * N) // BT + E
    RP = NBLK * BT
    xf = x.reshape(N, D)

    coefs = _routing(xf, Wg, routing_bias)

    sel = coefs > 0.0
    seli = sel.astype(jnp.int32)
    slot = jnp.cumsum(seli, axis=1) - seli
    oneh = (slot[:, None, :] == jnp.arange(TOP_K)[None, :, None]) \
        & sel[:, None, :]
    e4 = (oneh * jnp.arange(E)[None, None, :]).sum(-1)
    w4 = jnp.where(oneh, coefs[:, None, :], 0.0).sum(-1)
    valid4 = oneh.any(-1)
    rk = jnp.cumsum(seli, axis=0) - seli
    counts = seli.sum(0)
    nblk = (counts + BT - 1) // BT
    cumnb = jnp.cumsum(nblk)
    pad_off = (cumnb - nblk) * BT
    rk4 = jnp.take_along_axis(rk, e4, axis=1)
    inv4 = pad_off[e4] + rk4
    tok = jnp.broadcast_to(jnp.arange(N)[:, None], (N, TOP_K))
    scat_idx = jnp.where(valid4, inv4, RP)
    src_token = jnp.zeros((RP,), jnp.int32).at[scat_idx.reshape(-1)].set(
        tok.reshape(-1))
    flat_inv = inv4.T.reshape(-1)
    bid = jnp.arange(NBLK)
    block_expert = jnp.minimum(
        (bid[:, None] >= cumnb[None, :]).sum(1), E - 1).astype(jnp.int32)

    W1T = W1.swapaxes(1, 2).astype(jnp.bfloat16)
    W2T = W2.swapaxes(1, 2).astype(jnp.bfloat16)
    W3T = W3.swapaxes(1, 2).astype(jnp.bfloat16)
    WsaT = Ws1.T.astype(jnp.bfloat16)
    WsbT = Ws2.T.astype(jnp.bfloat16)
    WscT = Ws3.T.astype(jnp.bfloat16)

    Dp = D // 2
    x_pack = lax.bitcast_convert_type(
        xf.astype(jnp.bfloat16).reshape(N, Dp, 2), jnp.int32)
    xs_pack = _sc_gather(RP, Dp, jnp.int32)(x_pack, src_token)
    x_sorted = lax.bitcast_convert_type(
        xs_pack, jnp.bfloat16).reshape(RP, D)

    y = _grouped(x_sorted, block_expert, W1T, W2T, W3T, BT, NBLK)

    y_pack = lax.bitcast_convert_type(y.reshape(RP, Dp, 2), jnp.int32)
    yg_pack = _sc_gather(TOP_K * N, Dp, jnp.int32)(y_pack, flat_inv)
    yg = lax.bitcast_convert_type(
        yg_pack, jnp.bfloat16).reshape(TOP_K, N, D)

    out = _combine(xf, WsaT, WsbT, WscT, yg, w4)
    return out.reshape(B, S, D)


# speedup vs baseline: 2.5487x; 2.5487x over previous
import functools

import jax
import jax.numpy as jnp
from jax import lax
from jax.experimental import pallas as pl
from jax.experimental.pallas import tpu as pltpu
from jax.experimental.pallas import tpu_sc as plsc

E = 8
TOP_K = 4
_NN = (((1,), (0,)), ((), ()))


def _pack_bf16(xb):
    Dp = xb.shape[1] // 2
    lo = lax.bitcast_convert_type(xb[:, :Dp], jnp.uint16).astype(jnp.uint32)
    hi = lax.bitcast_convert_type(xb[:, Dp:], jnp.uint16).astype(jnp.uint32)
    return lax.bitcast_convert_type(lo | (hi << 16), jnp.int32)


def _unpack_bf16(w):
    wu = lax.bitcast_convert_type(w, jnp.uint32)
    lo = lax.bitcast_convert_type((wu & 0xFFFF).astype(jnp.uint16),
                                  jnp.bfloat16)
    hi = lax.bitcast_convert_type((wu >> 16).astype(jnp.uint16),
                                  jnp.bfloat16)
    return jnp.concatenate([lo, hi], axis=1)


def _route_body(x_ref, wgt_ref, bias_ref, coef_ref, xp_ref, *, T):
    xb = x_ref[...].astype(jnp.bfloat16)
    xp_ref[...] = _pack_bf16(xb)
    g = lax.dot_general(xb, wgt_ref[...], _NN,
                        preferred_element_type=jnp.float32) + bias_ref[...]
    lane = lax.broadcasted_iota(jnp.int32, (T, E), 1)
    cnt = jnp.zeros((T, E), jnp.int32)
    for jj in range(E):
        gj = g[:, jj:jj + 1]
        above = (gj > g) | ((gj == g) & (jj < lane))
        cnt = cnt + above.astype(jnp.int32)
    sel = cnt < TOP_K
    m = jnp.max(g, axis=1, keepdims=True)
    p = jnp.where(sel, jnp.exp(g - m), 0.0)
    coef_ref[...] = p / jnp.sum(p, axis=1, keepdims=True)


def _routing(xf, Wg, bias):
    N, D = xf.shape
    T = min(1024, N)
    return pl.pallas_call(
        functools.partial(_route_body, T=T),
        grid=(N // T,),
        in_specs=[
            pl.BlockSpec((T, D), lambda tb: (tb, 0)),
            pl.BlockSpec((D, E), lambda tb: (0, 0)),
            pl.BlockSpec((1, E), lambda tb: (0, 0)),
        ],
        out_specs=[
            pl.BlockSpec((T, E), lambda tb: (tb, 0)),
            pl.BlockSpec((T, D // 2), lambda tb: (tb, 0)),
        ],
        out_shape=[
            jax.ShapeDtypeStruct((N, E), jnp.float32),
            jax.ShapeDtypeStruct((N, D // 2), jnp.int32),
        ],
    )(xf, Wg.T.astype(jnp.bfloat16), bias.reshape(1, E).astype(jnp.float32))


def _sc_gather(R, C, dtype):
    mesh = plsc.VectorSubcoreMesh(core_axis_name="c", subcore_axis_name="s")
    rows_per_w = R // 32
    CH = 128
    n_ch = rows_per_w // CH

    @functools.partial(
        pl.kernel, mesh=mesh,
        out_type=jax.ShapeDtypeStruct((R, C), dtype),
        scratch_types=[
            pltpu.VMEM((CH,), jnp.int32),
            pltpu.VMEM((CH, C), dtype),
            pltpu.SemaphoreType.DMA,
        ],
    )
    def gather_k(table_hbm, idx_hbm, out_hbm, idx_v, rows_v, sem):
        wid = lax.axis_index("s") * 2 + lax.axis_index("c")
        base = wid * rows_per_w

        def chunk(g, carry):
            off = base + g * CH
            pltpu.sync_copy(idx_hbm.at[pl.ds(off, CH)], idx_v)
            pltpu.async_copy(table_hbm.at[idx_v], rows_v, sem).wait()
            pltpu.sync_copy(rows_v, out_hbm.at[pl.ds(off, CH)])
            return carry

        lax.fori_loop(0, n_ch, chunk, 0)

    return gather_k


def _group_body(be_ref, xs_ref, w1_ref, w2_ref, w3_ref, y_ref):
    xb = _unpack_bf16(xs_ref[...])
    h = lax.dot_general(xb, w1_ref[0], _NN, preferred_element_type=jnp.float32)
    h = h * jax.nn.sigmoid(h)
    v = lax.dot_general(xb, w2_ref[0], _NN, preferred_element_type=jnp.float32)
    hv = (h * v).astype(jnp.bfloat16)
    y = lax.dot_general(hv, w3_ref[0], _NN,
                        preferred_element_type=jnp.float32).astype(jnp.bfloat16)
    y_ref[...] = _pack_bf16(y)


def _grouped(xs_pack, block_expert, W1T, W2T, W3T, BT, NBLK):
    _, Dp = xs_pack.shape
    _, D, H = W1T.shape
    grid_spec = pltpu.PrefetchScalarGridSpec(
        num_scalar_prefetch=1,
        grid=(NBLK,),
        in_specs=[
            pl.BlockSpec((BT, Dp), lambda i, be: (i, 0)),
            pl.BlockSpec((1, D, H), lambda i, be: (be[i], 0, 0)),
            pl.BlockSpec((1, D, H), lambda i, be: (be[i], 0, 0)),
            pl.BlockSpec((1, H, D), lambda i, be: (be[i], 0, 0)),
        ],
        out_specs=pl.BlockSpec((BT, Dp), lambda i, be: (i, 0)),
    )
    return pl.pallas_call(
        _group_body,
        grid_spec=grid_spec,
        out_shape=jax.ShapeDtypeStruct((NBLK * BT, Dp), jnp.int32),
        compiler_params=pltpu.CompilerParams(
            dimension_semantics=("arbitrary",)),
    )(block_expert, xs_pack, W1T, W2T, W3T)


def _comb_body(x_ref, wsa_ref, wsb_ref, wsc_ref, yg_ref, w4_ref, out_ref):
    xb = x_ref[...].astype(jnp.bfloat16)
    h = lax.dot_general(xb, wsa_ref[...], _NN,
                        preferred_element_type=jnp.float32)
    h = h * jax.nn.sigmoid(h)
    v = lax.dot_general(xb, wsb_ref[...], _NN,
                        preferred_element_type=jnp.float32)
    hv = (h * v).astype(jnp.bfloat16)
    acc = lax.dot_general(hv, wsc_ref[...], _NN,
                          preferred_element_type=jnp.float32)
    for s in range(TOP_K):
        ys = _unpack_bf16(yg_ref[s])
        acc = acc + ys.astype(jnp.float32) * w4_ref[:, s:s + 1]
    out_ref[...] = acc


def _combine(xf, WsaT, WsbT, WscT, yg, w4):
    N, D = xf.shape
    _, H = WsaT.shape
    T = min(512, N)
    return pl.pallas_call(
        _comb_body,
        grid=(N // T,),
        in_specs=[
            pl.BlockSpec((T, D), lambda tb: (tb, 0)),
            pl.BlockSpec((D, H), lambda tb: (0, 0)),
            pl.BlockSpec((D, H), lambda tb: (0, 0)),
            pl.BlockSpec((H, D), lambda tb: (0, 0)),
            pl.BlockSpec((TOP_K, T, D // 2), lambda tb: (0, tb, 0)),
            pl.BlockSpec((T, TOP_K), lambda tb: (tb, 0)),
        ],
        out_specs=pl.BlockSpec((T, D), lambda tb: (tb, 0)),
        out_shape=jax.ShapeDtypeStruct((N, D), jnp.float32),
    )(xf, WsaT, WsbT, WscT, yg, w4)


def kernel(x, Wg, W1, W2, W3, Ws1, Ws2, Ws3, routing_bias):
    B, S, D = x.shape
    _, H, _ = W1.shape
    N = B * S
    BT = 512
    NBLK = (TOP_K * N) // BT + E
    RP = NBLK * BT
    xf = x.reshape(N, D)

    coefs, x_pack = _routing(xf, Wg, routing_bias)

    sel = coefs > 0.0
    seli = sel.astype(jnp.int32)
    slot = jnp.cumsum(seli, axis=1) - seli
    oneh = (slot[:, None, :] == jnp.arange(TOP_K)[None, :, None]) \
        & sel[:, None, :]
    e4 = (oneh * jnp.arange(E)[None, None, :]).sum(-1)
    w4 = jnp.where(oneh, coefs[:, None, :], 0.0).sum(-1)
    valid4 = oneh.any(-1)
    rk = jnp.cumsum(seli, axis=0) - seli
    counts = seli.sum(0)
    nblk = (counts + BT - 1) // BT
    cumnb = jnp.cumsum(nblk)
    pad_off = (cumnb - nblk) * BT
    rk4 = jnp.take_along_axis(rk, e4, axis=1)
    inv4 = pad_off[e4] + rk4
    tok = jnp.broadcast_to(jnp.arange(N)[:, None], (N, TOP_K))
    scat_idx = jnp.where(valid4, inv4, RP)
    src_token = jnp.zeros((RP,), jnp.int32).at[scat_idx.reshape(-1)].set(
        tok.reshape(-1))
    flat_inv = inv4.T.reshape(-1)
    bid = jnp.arange(NBLK)
    block_expert = jnp.minimum(
        (bid[:, None] >= cumnb[None, :]).sum(1), E - 1).astype(jnp.int32)

    W1T = W1.swapaxes(1, 2).astype(jnp.bfloat16)
    W2T = W2.swapaxes(1, 2).astype(jnp.bfloat16)
    W3T = W3.swapaxes(1, 2).astype(jnp.bfloat16)
    WsaT = Ws1.T.astype(jnp.bfloat16)
    WsbT = Ws2.T.astype(jnp.bfloat16)
    WscT = Ws3.T.astype(jnp.bfloat16)

    Dp = D // 2
    xs_pack = _sc_gather(RP, Dp, jnp.int32)(x_pack, src_token)

    y_pack = _grouped(xs_pack, block_expert, W1T, W2T, W3T, BT, NBLK)

    yg_pack = _sc_gather(TOP_K * N, Dp, jnp.int32)(y_pack, flat_inv)

    out = _combine(xf, WsaT, WsbT, WscT,
                   yg_pack.reshape(TOP_K, N, Dp), w4)
    return out.reshape(B, S, D)


# speedup vs baseline: 3.6661x; 1.4384x over previous
import functools

import jax
import jax.numpy as jnp
from jax import lax
from jax.experimental import pallas as pl
from jax.experimental.pallas import tpu as pltpu
from jax.experimental.pallas import tpu_sc as plsc

E = 8
TOP_K = 4
_NN = (((1,), (0,)), ((), ()))


def _pack_bf16(xb):
    Dp = xb.shape[1] // 2
    lo = lax.bitcast_convert_type(xb[:, :Dp], jnp.uint16).astype(jnp.uint32)
    hi = lax.bitcast_convert_type(xb[:, Dp:], jnp.uint16).astype(jnp.uint32)
    return lax.bitcast_convert_type(lo | (hi << 16), jnp.int32)


def _unpack_bf16(w):
    wu = lax.bitcast_convert_type(w, jnp.uint32)
    lo = lax.bitcast_convert_type((wu & 0xFFFF).astype(jnp.uint16),
                                  jnp.bfloat16)
    hi = lax.bitcast_convert_type((wu >> 16).astype(jnp.uint16),
                                  jnp.bfloat16)
    return jnp.concatenate([lo, hi], axis=1)


def _route_body(x_ref, wgt_ref, bias_ref, coef_ref, xp_ref, *, T):
    xb = x_ref[...].astype(jnp.bfloat16)
    xp_ref[...] = _pack_bf16(xb)
    g = lax.dot_general(xb, wgt_ref[...], _NN,
                        preferred_element_type=jnp.float32) + bias_ref[...]
    lane = lax.broadcasted_iota(jnp.int32, (T, E), 1)
    cnt = jnp.zeros((T, E), jnp.int32)
    for jj in range(E):
        gj = g[:, jj:jj + 1]
        above = (gj > g) | ((gj == g) & (jj < lane))
        cnt = cnt + above.astype(jnp.int32)
    sel = cnt < TOP_K
    m = jnp.max(g, axis=1, keepdims=True)
    p = jnp.where(sel, jnp.exp(g - m), 0.0)
    coef_ref[...] = p / jnp.sum(p, axis=1, keepdims=True)


def _routing(xf, Wg, bias):
    N, D = xf.shape
    T = min(1024, N)
    return pl.pallas_call(
        functools.partial(_route_body, T=T),
        grid=(N // T,),
        in_specs=[
            pl.BlockSpec((T, D), lambda tb: (tb, 0)),
            pl.BlockSpec((D, E), lambda tb: (0, 0)),
            pl.BlockSpec((1, E), lambda tb: (0, 0)),
        ],
        out_specs=[
            pl.BlockSpec((T, E), lambda tb: (tb, 0)),
            pl.BlockSpec((T, D // 2), lambda tb: (tb, 0)),
        ],
        out_shape=[
            jax.ShapeDtypeStruct((N, E), jnp.float32),
            jax.ShapeDtypeStruct((N, D // 2), jnp.int32),
        ],
    )(xf, Wg.T.astype(jnp.bfloat16), bias.reshape(1, E).astype(jnp.float32))


def _sc_gather(R, C, dtype):
    mesh = plsc.VectorSubcoreMesh(core_axis_name="c", subcore_axis_name="s")
    rows_per_w = R // 32
    CH = 128
    n_ch = rows_per_w // CH

    @functools.partial(
        pl.kernel, mesh=mesh,
        out_type=jax.ShapeDtypeStruct((R, C), dtype),
        scratch_types=[
            pltpu.VMEM((CH,), jnp.int32),
            pltpu.VMEM((CH, C), dtype),
            pltpu.SemaphoreType.DMA,
        ],
    )
    def gather_k(table_hbm, idx_hbm, out_hbm, idx_v, rows_v, sem):
        wid = lax.axis_index("s") * 2 + lax.axis_index("c")
        base = wid * rows_per_w

        def chunk(g, carry):
            off = base + g * CH
            pltpu.sync_copy(idx_hbm.at[pl.ds(off, CH)], idx_v)
            pltpu.async_copy(table_hbm.at[idx_v], rows_v, sem).wait()
            pltpu.sync_copy(rows_v, out_hbm.at[pl.ds(off, CH)])
            return carry

        lax.fori_loop(0, n_ch, chunk, 0)

    return gather_k


def _sc_dispatch(N, C, RP):
    mesh = plsc.VectorSubcoreMesh(core_axis_name="c", subcore_axis_name="s")
    tok_per_w = N // 32
    CH = 128
    n_ch = tok_per_w // CH

    @functools.partial(
        pl.kernel, mesh=mesh,
        out_type=jax.ShapeDtypeStruct((RP, C), jnp.int32),
        scratch_types=[
            pltpu.VMEM((n_ch * TOP_K, CH), jnp.int32),
            pltpu.VMEM((n_ch, CH, C), jnp.int32),
            pltpu.SemaphoreType.DMA,
        ],
    )
    def disp_k(xp_hbm, inv_hbm, out_hbm, idx_v, rows_v, sem):
        wid = lax.axis_index("s") * 2 + lax.axis_index("c")
        base = wid * tok_per_w
        for g in range(n_ch):
            pltpu.sync_copy(xp_hbm.at[pl.ds(base + g * CH, CH)], rows_v.at[g])
        copies = []
        for g in range(n_ch):
            for s in range(TOP_K):
                j = g * TOP_K + s
                pltpu.sync_copy(inv_hbm.at[pl.ds(s * N + base + g * CH, CH)],
                                idx_v.at[j])
                copies.append(
                    pltpu.async_copy(rows_v.at[g], out_hbm.at[idx_v.at[j]],
                                     sem))
        for c in copies:
            c.wait()

    return disp_k


def _group_body(be_ref, xs_ref, w1_ref, w2_ref, w3_ref, y_ref):
    xb = _unpack_bf16(xs_ref[...])
    h = lax.dot_general(xb, w1_ref[0], _NN, preferred_element_type=jnp.float32)
    h = h * jax.nn.sigmoid(h)
    v = lax.dot_general(xb, w2_ref[0], _NN, preferred_element_type=jnp.float32)
    hv = (h * v).astype(jnp.bfloat16)
    y = lax.dot_general(hv, w3_ref[0], _NN,
                        preferred_element_type=jnp.float32).astype(jnp.bfloat16)
    y_ref[...] = _pack_bf16(y)


def _grouped(xs_pack, block_expert, W1T, W2T, W3T, BT, NBLK):
    _, Dp = xs_pack.shape
    _, D, H = W1T.shape
    grid_spec = pltpu.PrefetchScalarGridSpec(
        num_scalar_prefetch=1,
        grid=(NBLK,),
        in_specs=[
            pl.BlockSpec((BT, Dp), lambda i, be: (i, 0)),
            pl.BlockSpec((1, D, H), lambda i, be: (be[i], 0, 0)),
            pl.BlockSpec((1, D, H), lambda i, be: (be[i], 0, 0)),
            pl.BlockSpec((1, H, D), lambda i, be: (be[i], 0, 0)),
        ],
        out_specs=pl.BlockSpec((BT, Dp), lambda i, be: (i, 0)),
    )
    return pl.pallas_call(
        _group_body,
        grid_spec=grid_spec,
        out_shape=jax.ShapeDtypeStruct((NBLK * BT, Dp), jnp.int32),
        compiler_params=pltpu.CompilerParams(
            dimension_semantics=("arbitrary",)),
    )(block_expert, xs_pack, W1T, W2T, W3T)


def _comb_body(x_ref, wsa_ref, wsb_ref, wsc_ref, yg_ref, w4_ref, out_ref):
    xb = x_ref[...].astype(jnp.bfloat16)
    h = lax.dot_general(xb, wsa_ref[...], _NN,
                        preferred_element_type=jnp.float32)
    h = h * jax.nn.sigmoid(h)
    v = lax.dot_general(xb, wsb_ref[...], _NN,
                        preferred_element_type=jnp.float32)
    hv = (h * v).astype(jnp.bfloat16)
    acc = lax.dot_general(hv, wsc_ref[...], _NN,
                          preferred_element_type=jnp.float32)
    for s in range(TOP_K):
        ys = _unpack_bf16(yg_ref[s])
        c = w4_ref[:, s:s + 1]
        acc = acc + jnp.where(c > 0.0, ys.astype(jnp.float32) * c, 0.0)
    out_ref[...] = acc


def _combine(xf, WsaT, WsbT, WscT, yg, w4):
    N, D = xf.shape
    _, H = WsaT.shape
    T = min(512, N)
    return pl.pallas_call(
        _comb_body,
        grid=(N // T,),
        in_specs=[
            pl.BlockSpec((T, D), lambda tb: (tb, 0)),
            pl.BlockSpec((D, H), lambda tb: (0, 0)),
            pl.BlockSpec((D, H), lambda tb: (0, 0)),
            pl.BlockSpec((H, D), lambda tb: (0, 0)),
            pl.BlockSpec((TOP_K, T, D // 2), lambda tb: (0, tb, 0)),
            pl.BlockSpec((T, TOP_K), lambda tb: (tb, 0)),
        ],
        out_specs=pl.BlockSpec((T, D), lambda tb: (tb, 0)),
        out_shape=jax.ShapeDtypeStruct((N, D), jnp.float32),
    )(xf, WsaT, WsbT, WscT, yg, w4)


def kernel(x, Wg, W1, W2, W3, Ws1, Ws2, Ws3, routing_bias):
    B, S, D = x.shape
    _, H, _ = W1.shape
    N = B * S
    BT = 512
    NBLK = (TOP_K * N) // BT + E
    RP = NBLK * BT
    xf = x.reshape(N, D)

    coefs, x_pack = _routing(xf, Wg, routing_bias)

    sel = coefs > 0.0
    seli = sel.astype(jnp.int32)
    slot = jnp.cumsum(seli, axis=1) - seli
    oneh = (slot[:, None, :] == jnp.arange(TOP_K)[None, :, None]) \
        & sel[:, None, :]
    e4 = (oneh * jnp.arange(E)[None, None, :]).sum(-1)
    w4 = jnp.where(oneh, coefs[:, None, :], 0.0).sum(-1)
    valid4 = oneh.any(-1)
    rk = jnp.cumsum(seli, axis=0) - seli
    counts = seli.sum(0)
    nblk = (counts + BT - 1) // BT
    cumnb = jnp.cumsum(nblk)
    pad_off = (cumnb - nblk) * BT
    rk4 = jnp.take_along_axis(rk, e4, axis=1)
    inv4 = pad_off[e4] + rk4
    inv4 = jnp.where(valid4, inv4, RP - 1)
    flat_inv = inv4.T.reshape(-1)
    bid = jnp.arange(NBLK)
    block_expert = jnp.minimum(
        (bid[:, None] >= cumnb[None, :]).sum(1), E - 1).astype(jnp.int32)

    W1T = W1.swapaxes(1, 2).astype(jnp.bfloat16)
    W2T = W2.swapaxes(1, 2).astype(jnp.bfloat16)
    W3T = W3.swapaxes(1, 2).astype(jnp.bfloat16)
    WsaT = Ws1.T.astype(jnp.bfloat16)
    WsbT = Ws2.T.astype(jnp.bfloat16)
    WscT = Ws3.T.astype(jnp.bfloat16)

    Dp = D // 2
    xs_pack = _sc_dispatch(N, Dp, RP)(x_pack, flat_inv)

    y_pack = _grouped(xs_pack, block_expert, W1T, W2T, W3T, BT, NBLK)

    yg_pack = _sc_gather(TOP_K * N, Dp, jnp.int32)(y_pack, flat_inv)

    out = _combine(xf, WsaT, WsbT, WscT,
                   yg_pack.reshape(TOP_K, N, Dp), w4)
    return out.reshape(B, S, D)


# speedup vs baseline: 3.7524x; 1.0236x over previous
import functools

import jax
import jax.numpy as jnp
from jax import lax
from jax.experimental import pallas as pl
from jax.experimental.pallas import tpu as pltpu
from jax.experimental.pallas import tpu_sc as plsc

E = 8
TOP_K = 4
_NN = (((1,), (0,)), ((), ()))


def _pack_bf16(xb):
    Dp = xb.shape[1] // 2
    lo = lax.bitcast_convert_type(xb[:, :Dp], jnp.uint16).astype(jnp.uint32)
    hi = lax.bitcast_convert_type(xb[:, Dp:], jnp.uint16).astype(jnp.uint32)
    return lax.bitcast_convert_type(lo | (hi << 16), jnp.int32)


def _unpack_bf16(w):
    wu = lax.bitcast_convert_type(w, jnp.uint32)
    lo = lax.bitcast_convert_type((wu & 0xFFFF).astype(jnp.uint16),
                                  jnp.bfloat16)
    hi = lax.bitcast_convert_type((wu >> 16).astype(jnp.uint16),
                                  jnp.bfloat16)
    return jnp.concatenate([lo, hi], axis=1)


def _route_body(x_ref, wgt_ref, bias_ref, xp_ref, w4_ref, e4_ref, rk4_ref,
                counts_ref, carry_ref, *, T):
    tb = pl.program_id(0)
    xb = x_ref[...].astype(jnp.bfloat16)
    xp_ref[...] = _pack_bf16(xb)
    g = lax.dot_general(xb, wgt_ref[...], _NN,
                        preferred_element_type=jnp.float32) + bias_ref[...]
    lane = lax.broadcasted_iota(jnp.int32, (T, E), 1)
    cnt = jnp.zeros((T, E), jnp.int32)
    for jj in range(E):
        gj = g[:, jj:jj + 1]
        above = (gj > g) | ((gj == g) & (jj < lane))
        cnt = cnt + above.astype(jnp.int32)
    sel = cnt < TOP_K
    m = jnp.max(g, axis=1, keepdims=True)
    p = jnp.where(sel, jnp.exp(g - m), 0.0)
    coef = p / jnp.sum(p, axis=1, keepdims=True)

    seli = sel.astype(jnp.int32)
    csum = seli
    row = lax.broadcasted_iota(jnp.int32, (T, E), 0)
    sh = 1
    while sh < T:
        rolled = pltpu.roll(csum, sh, 0)
        csum = csum + jnp.where(row >= sh, rolled, 0)
        sh *= 2
    @pl.when(tb == 0)
    def _():
        carry_ref[...] = jnp.zeros((1, E), jnp.int32)
    carry = carry_ref[...]
    rk = csum - seli + carry
    @pl.when(tb == 0)
    def _():
        counts_ref[...] = jnp.zeros((1, E), jnp.int32)
    counts_ref[...] += csum[T - 1:T, :]
    carry_ref[...] = carry + csum[T - 1:T, :]

    w4s, e4s, rk4s = [], [], []
    for s in range(TOP_K):
        oh = cnt == s
        e4s.append(jnp.sum(jnp.where(oh, lane, 0), axis=1, keepdims=True))
        w4s.append(jnp.sum(jnp.where(oh, coef, 0.0), axis=1, keepdims=True))
        rk4s.append(jnp.sum(jnp.where(oh, rk, 0), axis=1, keepdims=True))
    w4_ref[...] = jnp.concatenate(w4s, axis=1)
    e4_ref[...] = jnp.concatenate(e4s, axis=1)
    rk4_ref[...] = jnp.concatenate(rk4s, axis=1)


def _routing(xf, Wg, bias):
    N, D = xf.shape
    T = min(1024, N)
    return pl.pallas_call(
        functools.partial(_route_body, T=T),
        grid=(N // T,),
        in_specs=[
            pl.BlockSpec((T, D), lambda tb: (tb, 0)),
            pl.BlockSpec((D, E), lambda tb: (0, 0)),
            pl.BlockSpec((1, E), lambda tb: (0, 0)),
        ],
        out_specs=[
            pl.BlockSpec((T, D // 2), lambda tb: (tb, 0)),
            pl.BlockSpec((T, TOP_K), lambda tb: (tb, 0)),
            pl.BlockSpec((T, TOP_K), lambda tb: (tb, 0)),
            pl.BlockSpec((T, TOP_K), lambda tb: (tb, 0)),
            pl.BlockSpec((1, E), lambda tb: (0, 0)),
        ],
        out_shape=[
            jax.ShapeDtypeStruct((N, D // 2), jnp.int32),
            jax.ShapeDtypeStruct((N, TOP_K), jnp.float32),
            jax.ShapeDtypeStruct((N, TOP_K), jnp.int32),
            jax.ShapeDtypeStruct((N, TOP_K), jnp.int32),
            jax.ShapeDtypeStruct((1, E), jnp.int32),
        ],
        scratch_shapes=[pltpu.VMEM((1, E), jnp.int32)],
        compiler_params=pltpu.CompilerParams(
            dimension_semantics=("arbitrary",)),
    )(xf, Wg.T.astype(jnp.bfloat16), bias.reshape(1, E).astype(jnp.float32))


def _sc_gather(R, C, dtype):
    mesh = plsc.VectorSubcoreMesh(core_axis_name="c", subcore_axis_name="s")
    rows_per_w = R // 32
    CH = 128
    n_ch = rows_per_w // CH

    @functools.partial(
        pl.kernel, mesh=mesh,
        out_type=jax.ShapeDtypeStruct((R, C), dtype),
        scratch_types=[
            pltpu.VMEM((CH,), jnp.int32),
            pltpu.VMEM((CH, C), dtype),
            pltpu.SemaphoreType.DMA,
        ],
    )
    def gather_k(table_hbm, idx_hbm, out_hbm, idx_v, rows_v, sem):
        wid = lax.axis_index("s") * 2 + lax.axis_index("c")
        base = wid * rows_per_w

        def chunk(g, carry):
            off = base + g * CH
            pltpu.sync_copy(idx_hbm.at[pl.ds(off, CH)], idx_v)
            pltpu.async_copy(table_hbm.at[idx_v], rows_v, sem).wait()
            pltpu.sync_copy(rows_v, out_hbm.at[pl.ds(off, CH)])
            return carry

        lax.fori_loop(0, n_ch, chunk, 0)

    return gather_k


def _sc_dispatch(N, C, RP):
    mesh = plsc.VectorSubcoreMesh(core_axis_name="c", subcore_axis_name="s")
    tok_per_w = N // 32
    CH = 128
    n_ch = tok_per_w // CH

    @functools.partial(
        pl.kernel, mesh=mesh,
        out_type=jax.ShapeDtypeStruct((RP, C), jnp.int32),
        scratch_types=[
            pltpu.VMEM((n_ch * TOP_K, CH), jnp.int32),
            pltpu.VMEM((n_ch, CH, C), jnp.int32),
            pltpu.SemaphoreType.DMA,
        ],
    )
    def disp_k(xp_hbm, inv_hbm, out_hbm, idx_v, rows_v, sem):
        wid = lax.axis_index("s") * 2 + lax.axis_index("c")
        base = wid * tok_per_w
        for g in range(n_ch):
            pltpu.sync_copy(xp_hbm.at[pl.ds(base + g * CH, CH)], rows_v.at[g])
        copies = []
        for g in range(n_ch):
            for s in range(TOP_K):
                j = g * TOP_K + s
                pltpu.sync_copy(inv_hbm.at[pl.ds(s * N + base + g * CH, CH)],
                                idx_v.at[j])
                copies.append(
                    pltpu.async_copy(rows_v.at[g], out_hbm.at[idx_v.at[j]],
                                     sem))
        for c in copies:
            c.wait()

    return disp_k


def _group_body(be_ref, xs_ref, w1_ref, w2_ref, w3_ref, y_ref):
    xb = _unpack_bf16(xs_ref[...])
    h = lax.dot_general(xb, w1_ref[0], _NN, preferred_element_type=jnp.float32)
    h = h * jax.nn.sigmoid(h)
    v = lax.dot_general(xb, w2_ref[0], _NN, preferred_element_type=jnp.float32)
    hv = (h * v).astype(jnp.bfloat16)
    y = lax.dot_general(hv, w3_ref[0], _NN,
                        preferred_element_type=jnp.float32).astype(jnp.bfloat16)
    y_ref[...] = _pack_bf16(y)


def _grouped(xs_pack, block_expert, W1T, W2T, W3T, BT, NBLK):
    _, Dp = xs_pack.shape
    _, D, H = W1T.shape
    grid_spec = pltpu.PrefetchScalarGridSpec(
        num_scalar_prefetch=1,
        grid=(NBLK,),
        in_specs=[
            pl.BlockSpec((BT, Dp), lambda i, be: (i, 0)),
            pl.BlockSpec((1, D, H), lambda i, be: (be[i], 0, 0)),
            pl.BlockSpec((1, D, H), lambda i, be: (be[i], 0, 0)),
            pl.BlockSpec((1, H, D), lambda i, be: (be[i], 0, 0)),
        ],
        out_specs=pl.BlockSpec((BT, Dp), lambda i, be: (i, 0)),
    )
    return pl.pallas_call(
        _group_body,
        grid_spec=grid_spec,
        out_shape=jax.ShapeDtypeStruct((NBLK * BT, Dp), jnp.int32),
        compiler_params=pltpu.CompilerParams(
            dimension_semantics=("parallel",)),
    )(block_expert, xs_pack, W1T, W2T, W3T)


def _comb_body(x_ref, wsa_ref, wsb_ref, wsc_ref, yg_ref, w4_ref, out_ref):
    xb = x_ref[...].astype(jnp.bfloat16)
    h = lax.dot_general(xb, wsa_ref[...], _NN,
                        preferred_element_type=jnp.float32)
    h = h * jax.nn.sigmoid(h)
    v = lax.dot_general(xb, wsb_ref[...], _NN,
                        preferred_element_type=jnp.float32)
    hv = (h * v).astype(jnp.bfloat16)
    acc = lax.dot_general(hv, wsc_ref[...], _NN,
                          preferred_element_type=jnp.float32)
    for s in range(TOP_K):
        ys = _unpack_bf16(yg_ref[s])
        c = w4_ref[:, s:s + 1]
        acc = acc + jnp.where(c > 0.0, ys.astype(jnp.float32) * c, 0.0)
    out_ref[...] = acc


def _combine(xf, WsaT, WsbT, WscT, yg, w4):
    N, D = xf.shape
    _, H = WsaT.shape
    T = min(512, N)
    return pl.pallas_call(
        _comb_body,
        grid=(N // T,),
        in_specs=[
            pl.BlockSpec((T, D), lambda tb: (tb, 0)),
            pl.BlockSpec((D, H), lambda tb: (0, 0)),
            pl.BlockSpec((D, H), lambda tb: (0, 0)),
            pl.BlockSpec((H, D), lambda tb: (0, 0)),
            pl.BlockSpec((TOP_K, T, D // 2), lambda tb: (0, tb, 0)),
            pl.BlockSpec((T, TOP_K), lambda tb: (tb, 0)),
        ],
        out_specs=pl.BlockSpec((T, D), lambda tb: (tb, 0)),
        out_shape=jax.ShapeDtypeStruct((N, D), jnp.float32),
    )(xf, WsaT, WsbT, WscT, yg, w4)


def kernel(x, Wg, W1, W2, W3, Ws1, Ws2, Ws3, routing_bias):
    B, S, D = x.shape
    _, H, _ = W1.shape
    N = B * S
    BT = 512
    NBLK = (TOP_K * N) // BT + E
    RP = NBLK * BT
    xf = x.reshape(N, D)

    x_pack, w4, e4, rk4, counts = _routing(xf, Wg, routing_bias)

    counts = counts.reshape(E)
    nblk = (counts + BT - 1) // BT
    cumnb = jnp.cumsum(nblk)
    pad_off = (cumnb - nblk) * BT
    inv4 = pad_off[e4] + rk4
    flat_inv = inv4.T.reshape(-1)
    bid = jnp.arange(NBLK)
    block_expert = jnp.minimum(
        (bid[:, None] >= cumnb[None, :]).sum(1), E - 1).astype(jnp.int32)

    W1T = W1.swapaxes(1, 2).astype(jnp.bfloat16)
    W2T = W2.swapaxes(1, 2).astype(jnp.bfloat16)
    W3T = W3.swapaxes(1, 2).astype(jnp.bfloat16)
    WsaT = Ws1.T.astype(jnp.bfloat16)
    WsbT = Ws2.T.astype(jnp.bfloat16)
    WscT = Ws3.T.astype(jnp.bfloat16)

    Dp = D // 2
    xs_pack = _sc_dispatch(N, Dp, RP)(x_pack, flat_inv)

    y_pack = _grouped(xs_pack, block_expert, W1T, W2T, W3T, BT, NBLK)

    yg_pack = _sc_gather(TOP_K * N, Dp, jnp.int32)(y_pack, flat_inv)

    out = _combine(xf, WsaT, WsbT, WscT,
                   yg_pack.reshape(TOP_K, N, Dp), w4)
    return out.reshape(B, S, D)
